# baseline (speedup 1.0000x reference)
import functools

import jax
import jax.numpy as jnp
from jax.experimental import pallas as pl
import jax.experimental.pallas.tpu as pltpu

ROWS = 3
B, CP, HP, WP = 8, 8, 256, 256
H = HP + ROWS - 1
W = WP + ROWS - 1
C_CONST = 0.9
LORRIS = 0.25
HAMMER = 1.0
N = B * HP * WP
THRESH = C_CONST ** (1.0 / (H * W))

CHUNK = 64
NCHUNK = HP // CHUNK


def _loss_kernel(mu_ref, exp_ref, pr_ref, idx_ref, w1_ref, u0_ref, u1_ref,
                 v_ref, b_ref, loss_ref, acc_ref):
    bidx = pl.program_id(0)

    @pl.when(bidx == 0)
    def _init():
        for t in range(4):
            acc_ref[t] = 0.0

    cnt_acc = jnp.zeros((CHUNK, WP), dtype=jnp.float32)
    wid_acc = jnp.zeros((CHUNK, WP), dtype=jnp.float32)
    pen_acc = jnp.zeros((CHUNK, WP), dtype=jnp.float32)

    for chunk in range(NCHUNK):
        base = chunk * CHUNK
        rdc0 = (exp_ref[0, 0, base:base + CHUNK + 2, :]
                - mu_ref[0, 0, base:base + CHUNK + 2, :])
        rdc1 = (exp_ref[0, 1, base:base + CHUNK + 2, :]
                - mu_ref[0, 1, base:base + CHUNK + 2, :])
        t0 = [rdc0[:, j:j + WP] for j in range(ROWS)]
        t1 = [rdc1[:, j:j + WP] for j in range(ROWS)]
        idx = idx_ref[0, base:base + CHUNK, :]

        r = [jnp.full((CHUNK, WP), b_ref[0, c], dtype=jnp.float32)
             for c in range(4)]
        for ci in range(CP):
            p = pr_ref[0, ci, base:base + CHUNK, :]
            for c in range(4):
                r[c] += p * w1_ref[ci, c]

        piv0 = jnp.zeros((CHUNK, WP), dtype=jnp.float32)
        piv1 = jnp.zeros((CHUNK, WP), dtype=jnp.float32)
        for k in range(ROWS * ROWS):
            i, j = divmod(k, ROWS)
            s0 = t0[j][i:i + CHUNK, :]
            s1 = t1[j][i:i + CHUNK, :]
            m = idx == k
            piv0 = jnp.where(m, s0, piv0)
            piv1 = jnp.where(m, s1, piv1)
            for c in range(4):
                term = s0 * u0_ref[k, c] + s1 * u1_ref[k, c]
                r[c] += jnp.where(m, v_ref[k, c], term)

        r0, r1, r2, r3 = r
        full_in = (((piv0 - r0) >= 0.0) & ((piv1 - r1) >= 0.0)
                   & ((piv0 - r2) <= 0.0) & ((piv1 - r3) <= 0.0))
        cnt_acc += full_in.astype(jnp.float32)
        wid_acc += jnp.abs(r2 - r0) + jnp.abs(r3 - r1)
        over0 = jnp.maximum(piv0 - r2, 0.0)
        over1 = jnp.maximum(piv1 - r3, 0.0)
        under0 = jnp.maximum(r0 - piv0, 0.0)
        under1 = jnp.maximum(r1 - piv1, 0.0)
        pen_acc += (over0 * over0 + over1 * over1
                    + under0 * under0 + under1 * under1)

    acc_ref[0] += jnp.sum(cnt_acc)
    acc_ref[1] += jnp.sum(wid_acc)
    acc_ref[2] += jnp.sum(pen_acc)

    @pl.when(bidx == B - 1)
    def _fini():
        p_in = acc_ref[0] * (1.0 / N)
        penalty = acc_ref[2] * (HAMMER / (2.0 * N))
        loss = LORRIS * acc_ref[1] + jnp.where(p_in < THRESH, penalty, 0.0)
        loss_ref[0, 0] = loss


@functools.partial(jax.jit)
def _run(actual_mu, actual_pruning, expected, W1, W2, b, idx3):
    W2r = W2.reshape(ROWS * ROWS, 4, 4)
    U0 = W2r[:, 0, :] + W2r[:, 2, :]
    U1 = W2r[:, 1, :] + W2r[:, 3, :]
    V = W2r[:, 0, :] + W2r[:, 1, :] - W2r[:, 2, :] - W2r[:, 3, :]
    b2 = b.reshape(1, 4)

    smem = pl.BlockSpec(memory_space=pltpu.SMEM)
    out = pl.pallas_call(
        _loss_kernel,
        grid=(B,),
        in_specs=[
            pl.BlockSpec((1, 2, H, W), lambda i: (i, 0, 0, 0)),
            pl.BlockSpec((1, 2, H, W), lambda i: (i, 0, 0, 0)),
            pl.BlockSpec((1, CP, HP, WP), lambda i: (i, 0, 0, 0)),
            pl.BlockSpec((1, HP, WP), lambda i: (i, 0, 0)),
            smem, smem, smem, smem, smem,
        ],
        out_specs=pl.BlockSpec(memory_space=pltpu.SMEM),
        out_shape=jax.ShapeDtypeStruct((1, 1), jnp.float32),
        scratch_shapes=[pltpu.SMEM((4,), jnp.float32)],
        compiler_params=pltpu.CompilerParams(
            dimension_semantics=("arbitrary",)),
    )(actual_mu, expected, actual_pruning, idx3, W1, U0, U1, V, b2)
    return out.reshape(())


def kernel(actual_mu, actual_pruning, expected, W1, W2, b, index_choice):
    idx3 = index_choice.reshape(B, HP, WP)
    return _run(actual_mu, actual_pruning, expected, W1, W2, b, idx3)


# speedup vs baseline: 1.0026x; 1.0026x over previous
import functools

import jax
import jax.numpy as jnp
from jax.experimental import pallas as pl
import jax.experimental.pallas.tpu as pltpu

ROWS = 3
B, CP, HP, WP = 8, 8, 256, 256
H = HP + ROWS - 1
W = WP + ROWS - 1
C_CONST = 0.9
LORRIS = 0.25
HAMMER = 1.0
N = B * HP * WP
THRESH = C_CONST ** (1.0 / (H * W))

CHUNK = 16
NCHUNK = HP // CHUNK


def _loss_kernel(mu_ref, exp_ref, pr_ref, idx_ref, w1_ref, u0_ref, u1_ref,
                 v_ref, b_ref, loss_ref, acc_ref):
    bidx = pl.program_id(0)

    @pl.when(bidx == 0)
    def _init():
        for t in range(4):
            acc_ref[t] = 0.0

    cnt_acc = jnp.zeros((CHUNK, WP), dtype=jnp.float32)
    wid_acc = jnp.zeros((CHUNK, WP), dtype=jnp.float32)
    pen_acc = jnp.zeros((CHUNK, WP), dtype=jnp.float32)

    for chunk in range(NCHUNK):
        base = chunk * CHUNK
        rdc0 = (exp_ref[0, 0, base:base + CHUNK + 2, :]
                - mu_ref[0, 0, base:base + CHUNK + 2, :])
        rdc1 = (exp_ref[0, 1, base:base + CHUNK + 2, :]
                - mu_ref[0, 1, base:base + CHUNK + 2, :])
        t0 = [rdc0[:, j:j + WP] for j in range(ROWS)]
        t1 = [rdc1[:, j:j + WP] for j in range(ROWS)]
        idx = idx_ref[0, base:base + CHUNK, :]

        r = [jnp.full((CHUNK, WP), b_ref[0, c], dtype=jnp.float32)
             for c in range(4)]
        for ci in range(CP):
            p = pr_ref[0, ci, base:base + CHUNK, :]
            for c in range(4):
                r[c] += p * w1_ref[ci, c]

        piv0 = jnp.zeros((CHUNK, WP), dtype=jnp.float32)
        piv1 = jnp.zeros((CHUNK, WP), dtype=jnp.float32)
        for k in range(ROWS * ROWS):
            i, j = divmod(k, ROWS)
            s0 = t0[j][i:i + CHUNK, :]
            s1 = t1[j][i:i + CHUNK, :]
            m = idx == k
            piv0 = jnp.where(m, s0, piv0)
            piv1 = jnp.where(m, s1, piv1)
            for c in range(4):
                term = s0 * u0_ref[k, c] + s1 * u1_ref[k, c]
                r[c] += jnp.where(m, v_ref[k, c], term)

        r0, r1, r2, r3 = r
        full_in = (((piv0 - r0) >= 0.0) & ((piv1 - r1) >= 0.0)
                   & ((piv0 - r2) <= 0.0) & ((piv1 - r3) <= 0.0))
        cnt_acc += full_in.astype(jnp.float32)
        wid_acc += jnp.abs(r2 - r0) + jnp.abs(r3 - r1)
        over0 = jnp.maximum(piv0 - r2, 0.0)
        over1 = jnp.maximum(piv1 - r3, 0.0)
        under0 = jnp.maximum(r0 - piv0, 0.0)
        under1 = jnp.maximum(r1 - piv1, 0.0)
        pen_acc += (over0 * over0 + over1 * over1
                    + under0 * under0 + under1 * under1)

    acc_ref[0] += jnp.sum(cnt_acc)
    acc_ref[1] += jnp.sum(wid_acc)
    acc_ref[2] += jnp.sum(pen_acc)

    @pl.when(bidx == B - 1)
    def _fini():
        p_in = acc_ref[0] * (1.0 / N)
        penalty = acc_ref[2] * (HAMMER / (2.0 * N))
        loss = LORRIS * acc_ref[1] + jnp.where(p_in < THRESH, penalty, 0.0)
        loss_ref[0, 0] = loss


@functools.partial(jax.jit)
def _run(actual_mu, actual_pruning, expected, W1, W2, b, idx3):
    W2r = W2.reshape(ROWS * ROWS, 4, 4)
    U0 = W2r[:, 0, :] + W2r[:, 2, :]
    U1 = W2r[:, 1, :] + W2r[:, 3, :]
    V = W2r[:, 0, :] + W2r[:, 1, :] - W2r[:, 2, :] - W2r[:, 3, :]
    b2 = b.reshape(1, 4)

    smem = pl.BlockSpec(memory_space=pltpu.SMEM)
    out = pl.pallas_call(
        _loss_kernel,
        grid=(B,),
        in_specs=[
            pl.BlockSpec((1, 2, H, W), lambda i: (i, 0, 0, 0)),
            pl.BlockSpec((1, 2, H, W), lambda i: (i, 0, 0, 0)),
            pl.BlockSpec((1, CP, HP, WP), lambda i: (i, 0, 0, 0)),
            pl.BlockSpec((1, HP, WP), lambda i: (i, 0, 0)),
            smem, smem, smem, smem, smem,
        ],
        out_specs=pl.BlockSpec(memory_space=pltpu.SMEM),
        out_shape=jax.ShapeDtypeStruct((1, 1), jnp.float32),
        scratch_shapes=[pltpu.SMEM((4,), jnp.float32)],
        compiler_params=pltpu.CompilerParams(
            dimension_semantics=("arbitrary",)),
    )(actual_mu, expected, actual_pruning, idx3, W1, U0, U1, V, b2)
    return out.reshape(())


def kernel(actual_mu, actual_pruning, expected, W1, W2, b, index_choice):
    idx3 = index_choice.reshape(B, HP, WP)
    return _run(actual_mu, actual_pruning, expected, W1, W2, b, idx3)


# speedup vs baseline: 1.0161x; 1.0135x over previous
import functools

import jax
import jax.numpy as jnp
from jax.experimental import pallas as pl
import jax.experimental.pallas.tpu as pltpu

ROWS = 3
B, CP, HP, WP = 8, 8, 256, 256
H = HP + ROWS - 1
W = WP + ROWS - 1
C_CONST = 0.9
LORRIS = 0.25
HAMMER = 1.0
N = B * HP * WP
THRESH = C_CONST ** (1.0 / (H * W))

CHUNK = 32
NCHUNK = HP // CHUNK


def _loss_kernel(mu_ref, exp_ref, pr_ref, idx_ref, w1_ref, u0_ref, u1_ref,
                 v_ref, b_ref, loss_ref, acc_ref):
    bidx = pl.program_id(0)

    @pl.when(bidx == 0)
    def _init():
        for t in range(4):
            acc_ref[t] = 0.0

    cnt_acc = jnp.zeros((CHUNK, WP), dtype=jnp.float32)
    wid_acc = jnp.zeros((CHUNK, WP), dtype=jnp.float32)
    pen_acc = jnp.zeros((CHUNK, WP), dtype=jnp.float32)

    for chunk in range(NCHUNK):
        base = chunk * CHUNK
        rdc0 = (exp_ref[0, 0, base:base + CHUNK + 2, :]
                - mu_ref[0, 0, base:base + CHUNK + 2, :])
        rdc1 = (exp_ref[0, 1, base:base + CHUNK + 2, :]
                - mu_ref[0, 1, base:base + CHUNK + 2, :])
        t0 = [rdc0[:, j:j + WP] for j in range(ROWS)]
        t1 = [rdc1[:, j:j + WP] for j in range(ROWS)]
        idx = idx_ref[0, base:base + CHUNK, :]

        r = [jnp.full((CHUNK, WP), b_ref[0, c], dtype=jnp.float32)
             for c in range(4)]
        for ci in range(CP):
            p = pr_ref[0, ci, base:base + CHUNK, :]
            for c in range(4):
                r[c] += p * w1_ref[ci, c]

        piv0 = jnp.zeros((CHUNK, WP), dtype=jnp.float32)
        piv1 = jnp.zeros((CHUNK, WP), dtype=jnp.float32)
        for k in range(ROWS * ROWS):
            i, j = divmod(k, ROWS)
            s0 = t0[j][i:i + CHUNK, :]
            s1 = t1[j][i:i + CHUNK, :]
            m = idx == k
            piv0 = jnp.where(m, s0, piv0)
            piv1 = jnp.where(m, s1, piv1)
            for c in range(4):
                term = s0 * u0_ref[k, c] + s1 * u1_ref[k, c]
                r[c] += jnp.where(m, v_ref[k, c], term)

        r0, r1, r2, r3 = r
        full_in = (((piv0 - r0) >= 0.0) & ((piv1 - r1) >= 0.0)
                   & ((piv0 - r2) <= 0.0) & ((piv1 - r3) <= 0.0))
        cnt_acc += full_in.astype(jnp.float32)
        wid_acc += jnp.abs(r2 - r0) + jnp.abs(r3 - r1)
        over0 = jnp.maximum(piv0 - r2, 0.0)
        over1 = jnp.maximum(piv1 - r3, 0.0)
        under0 = jnp.maximum(r0 - piv0, 0.0)
        under1 = jnp.maximum(r1 - piv1, 0.0)
        pen_acc += (over0 * over0 + over1 * over1
                    + under0 * under0 + under1 * under1)

    acc_ref[0] += jnp.sum(cnt_acc)
    acc_ref[1] += jnp.sum(wid_acc)
    acc_ref[2] += jnp.sum(pen_acc)

    @pl.when(bidx == B - 1)
    def _fini():
        p_in = acc_ref[0] * (1.0 / N)
        penalty = acc_ref[2] * (HAMMER / (2.0 * N))
        loss = LORRIS * acc_ref[1] + jnp.where(p_in < THRESH, penalty, 0.0)
        loss_ref[0, 0] = loss


@functools.partial(jax.jit)
def _run(actual_mu, actual_pruning, expected, W1, W2, b, idx3):
    W2r = W2.reshape(ROWS * ROWS, 4, 4)
    U0 = W2r[:, 0, :] + W2r[:, 2, :]
    U1 = W2r[:, 1, :] + W2r[:, 3, :]
    V = W2r[:, 0, :] + W2r[:, 1, :] - W2r[:, 2, :] - W2r[:, 3, :]
    b2 = b.reshape(1, 4)

    smem = pl.BlockSpec(memory_space=pltpu.SMEM)
    out = pl.pallas_call(
        _loss_kernel,
        grid=(B,),
        in_specs=[
            pl.BlockSpec((1, 2, H, W), lambda i: (i, 0, 0, 0)),
            pl.BlockSpec((1, 2, H, W), lambda i: (i, 0, 0, 0)),
            pl.BlockSpec((1, CP, HP, WP), lambda i: (i, 0, 0, 0)),
            pl.BlockSpec((1, HP, WP), lambda i: (i, 0, 0)),
            smem, smem, smem, smem, smem,
        ],
        out_specs=pl.BlockSpec(memory_space=pltpu.SMEM),
        out_shape=jax.ShapeDtypeStruct((1, 1), jnp.float32),
        scratch_shapes=[pltpu.SMEM((4,), jnp.float32)],
        compiler_params=pltpu.CompilerParams(
            dimension_semantics=("arbitrary",)),
    )(actual_mu, expected, actual_pruning, idx3, W1, U0, U1, V, b2)
    return out.reshape(())


def kernel(actual_mu, actual_pruning, expected, W1, W2, b, index_choice):
    idx3 = index_choice.reshape(B, HP, WP)
    return _run(actual_mu, actual_pruning, expected, W1, W2, b, idx3)


# speedup vs baseline: 1.0226x; 1.0064x over previous
import functools

import jax
import jax.numpy as jnp
from jax.experimental import pallas as pl
import jax.experimental.pallas.tpu as pltpu

ROWS = 3
B, CP, HP, WP = 8, 8, 256, 256
H = HP + ROWS - 1
W = WP + ROWS - 1
C_CONST = 0.9
LORRIS = 0.25
HAMMER = 1.0
N = B * HP * WP
THRESH = C_CONST ** (1.0 / (H * W))

CHUNK = 32
NCHUNK = HP // CHUNK


def _loss_kernel(mu_ref, exp_ref, pr_ref, idx_ref, w1_ref, u0_ref, u1_ref,
                 v_ref, b_ref, loss_ref, vacc_ref):
    bidx = pl.program_id(0)

    cnt_acc = jnp.zeros((CHUNK, WP), dtype=jnp.float32)
    wid_acc = jnp.zeros((CHUNK, WP), dtype=jnp.float32)
    pen_acc = jnp.zeros((CHUNK, WP), dtype=jnp.float32)

    for chunk in range(NCHUNK):
        base = chunk * CHUNK
        rdc0 = (exp_ref[0, 0, base:base + CHUNK + 2, :]
                - mu_ref[0, 0, base:base + CHUNK + 2, :])
        rdc1 = (exp_ref[0, 1, base:base + CHUNK + 2, :]
                - mu_ref[0, 1, base:base + CHUNK + 2, :])
        t0 = [rdc0[:, j:j + WP] for j in range(ROWS)]
        t1 = [rdc1[:, j:j + WP] for j in range(ROWS)]
        idx = idx_ref[0, base:base + CHUNK, :]

        r = [jnp.full((CHUNK, WP), b_ref[0, c], dtype=jnp.float32)
             for c in range(4)]
        for ci in range(CP):
            p = pr_ref[0, ci, base:base + CHUNK, :]
            for c in range(4):
                r[c] += p * w1_ref[ci, c]

        piv0 = jnp.zeros((CHUNK, WP), dtype=jnp.float32)
        piv1 = jnp.zeros((CHUNK, WP), dtype=jnp.float32)
        for k in range(ROWS * ROWS):
            i, j = divmod(k, ROWS)
            s0 = t0[j][i:i + CHUNK, :]
            s1 = t1[j][i:i + CHUNK, :]
            m = idx == k
            piv0 = jnp.where(m, s0, piv0)
            piv1 = jnp.where(m, s1, piv1)
            for c in range(4):
                term = s0 * u0_ref[k, c] + s1 * u1_ref[k, c]
                r[c] += jnp.where(m, v_ref[k, c], term)

        r0, r1, r2, r3 = r
        full_in = (((piv0 - r0) >= 0.0) & ((piv1 - r1) >= 0.0)
                   & ((piv0 - r2) <= 0.0) & ((piv1 - r3) <= 0.0))
        cnt_acc += full_in.astype(jnp.float32)
        wid_acc += jnp.abs(r2 - r0) + jnp.abs(r3 - r1)
        over0 = jnp.maximum(piv0 - r2, 0.0)
        over1 = jnp.maximum(piv1 - r3, 0.0)
        under0 = jnp.maximum(r0 - piv0, 0.0)
        under1 = jnp.maximum(r1 - piv1, 0.0)
        pen_acc += (over0 * over0 + over1 * over1
                    + under0 * under0 + under1 * under1)

    @pl.when(bidx == 0)
    def _init():
        vacc_ref[0] = cnt_acc
        vacc_ref[1] = wid_acc
        vacc_ref[2] = pen_acc

    @pl.when(bidx > 0)
    def _accum():
        vacc_ref[0] += cnt_acc
        vacc_ref[1] += wid_acc
        vacc_ref[2] += pen_acc

    @pl.when(bidx == B - 1)
    def _fini():
        p_in = jnp.sum(vacc_ref[0]) * (1.0 / N)
        penalty = jnp.sum(vacc_ref[2]) * (HAMMER / (2.0 * N))
        loss = (LORRIS * jnp.sum(vacc_ref[1])
                + jnp.where(p_in < THRESH, penalty, 0.0))
        loss_ref[0, 0] = loss


@functools.partial(jax.jit)
def _run(actual_mu, actual_pruning, expected, W1, W2, b, idx3):
    W2r = W2.reshape(ROWS * ROWS, 4, 4)
    U0 = W2r[:, 0, :] + W2r[:, 2, :]
    U1 = W2r[:, 1, :] + W2r[:, 3, :]
    V = W2r[:, 0, :] + W2r[:, 1, :] - W2r[:, 2, :] - W2r[:, 3, :]
    b2 = b.reshape(1, 4)

    smem = pl.BlockSpec(memory_space=pltpu.SMEM)
    out = pl.pallas_call(
        _loss_kernel,
        grid=(B,),
        in_specs=[
            pl.BlockSpec((1, 2, H, W), lambda i: (i, 0, 0, 0)),
            pl.BlockSpec((1, 2, H, W), lambda i: (i, 0, 0, 0)),
            pl.BlockSpec((1, CP, HP, WP), lambda i: (i, 0, 0, 0)),
            pl.BlockSpec((1, HP, WP), lambda i: (i, 0, 0)),
            smem, smem, smem, smem, smem,
        ],
        out_specs=pl.BlockSpec(memory_space=pltpu.SMEM),
        out_shape=jax.ShapeDtypeStruct((1, 1), jnp.float32),
        scratch_shapes=[pltpu.VMEM((3, CHUNK, WP), jnp.float32)],
        compiler_params=pltpu.CompilerParams(
            dimension_semantics=("arbitrary",)),
    )(actual_mu, expected, actual_pruning, idx3, W1, U0, U1, V, b2)
    return out.reshape(())


def kernel(actual_mu, actual_pruning, expected, W1, W2, b, index_choice):
    idx3 = index_choice.reshape(B, HP, WP)
    return _run(actual_mu, actual_pruning, expected, W1, W2, b, idx3)
